# baseline (speedup 1.0000x reference)
import functools

import jax
import jax.numpy as jnp
from jax import lax
from jax.experimental import pallas as pl
from jax.experimental.pallas import tpu as pltpu
from jax.experimental.pallas import tpu_sc as plsc

N = 10000
E = 320000
D = 128
K = 128
EPAD = 327680
NPAD = 10112
RPT = NPAD // 16
NBUF = 2
LOOK = 1
SPAN = 40


@functools.lru_cache(maxsize=None)
def _make_sc_aggregate(heavy_spans, light_spans):
    slots = (heavy_spans + light_spans) * 16
    assert slots * SPAN * K == EPAD
    mesh = plsc.VectorSubcoreMesh(core_axis_name="c", subcore_axis_name="s")

    @functools.partial(
        pl.kernel,
        out_type=jax.ShapeDtypeStruct((2, NPAD, D), jnp.float32),
        mesh=mesh,
        scratch_types=[
            pltpu.VMEM((SPAN, K), jnp.int32),
            pltpu.VMEM((SPAN, K), jnp.int32),
            [pltpu.VMEM((K, D), jnp.float32)] * NBUF,
            pltpu.VMEM_SHARED((NPAD, D), jnp.float32),
            [pltpu.SemaphoreType.DMA] * NBUF,
            [pltpu.SemaphoreType.DMA] * NBUF,
        ],
    )
    def sc_aggregate(table, src3, dst3, zeros, out, src_v, dst_v, bufs, acc,
                     gsems, ssems):
        c = lax.axis_index("c")
        s = lax.axis_index("s")
        tab = table
        rows = pl.ds(s * RPT, RPT)
        with jax.named_scope("zinit"):
            pltpu.sync_copy(zeros.at[rows], acc.at[rows])
            plsc.subcore_barrier()

        def wait_gather(j, b):
            pltpu.make_async_copy(tab.at[src_v.at[j]], bufs[b],
                                  gsems[b]).wait()

        def fire_scatter(j, b):
            pltpu.async_copy(bufs[b], acc.at[dst_v.at[j]], ssems[b], add=True)

        def wait_scatter(j, b):
            pltpu.make_async_copy(bufs[b], acc.at[dst_v.at[j]],
                                  ssems[b]).wait()

        def fire_gather(j, b):
            pltpu.async_copy(tab.at[src_v.at[j]], bufs[b], gsems[b])

        def steady(j, b):
            wait_gather(j, b)
            fire_scatter(j, b)
            bn = (b + LOOK) % NBUF
            wait_scatter(j + LOOK - NBUF, bn)
            fire_gather(j + LOOK, bn)

        def run_span(slot):
            pltpu.sync_copy(src3.at[slot], src_v)
            pltpu.sync_copy(dst3.at[slot], dst_v)
            for b in range(LOOK):
                pltpu.async_copy(tab.at[src_v.at[b]], bufs[b], gsems[b])

            for j in range(NBUF - LOOK):
                wait_gather(j, j)
                fire_scatter(j, j)
                fire_gather(j + LOOK, j + LOOK)

            start = NBUF - LOOK
            rem = (SPAN - NBUF) % NBUF
            for j in range(start, start + rem):
                steady(j, j % NBUF)

            def group(g, carry):
                for i in range(NBUF):
                    steady(start + rem + g * NBUF + i,
                           (start + rem + i) % NBUF)
                return carry

            lax.fori_loop(0, (SPAN - NBUF) // NBUF, group, 0)

            for j in range(SPAN - LOOK, SPAN):
                wait_gather(j, j % NBUF)
                fire_scatter(j, j % NBUF)
            for m in range(SPAN - NBUF, SPAN):
                wait_scatter(m, m % NBUF)

        with jax.named_scope("edges"):
            @pl.when(c == 0)
            def _():
                for sp in range(heavy_spans):
                    run_span(sp * 16 + s)

            @pl.when(c == 1)
            def _():
                for sp in range(light_spans):
                    run_span((heavy_spans + sp) * 16 + s)

        with jax.named_scope("outbar"):
            plsc.subcore_barrier()
        with jax.named_scope("outcp"):
            pltpu.sync_copy(acc.at[rows], out.at[c, rows])

    return sc_aggregate


def _tc1_body(x_ref, p_ref, wa_ref, ba_ref, wb_ref, bb_ref, ha_ref, hb_ref):
    sblk = x_ref[...] + p_ref[0] + p_ref[1]
    t = jnp.dot(sblk, wa_ref[...], preferred_element_type=jnp.float32)
    t = jnp.maximum(t + ba_ref[...], 0.0)
    u = jnp.dot(t, wb_ref[...], preferred_element_type=jnp.float32)
    u = jnp.maximum(u + bb_ref[...], 0.0)
    nrm = jnp.sqrt(jnp.sum(u * u, axis=1, keepdims=True))
    h = u / jnp.maximum(nrm, 1e-12)
    ha_ref[...] = h[:, :D]
    hb_ref[...] = h[:, D:]


def _tc2_body(ha_ref, hb_ref, pa_ref, pb_ref, w1a_ref, b1a_ref, w1b_ref,
              b1b_ref, wl1_ref, bl1_ref, wl2_ref, bl2_ref, out_ref):
    sa = ha_ref[...] + pa_ref[0] + pa_ref[1]
    sb = hb_ref[...] + pb_ref[0] + pb_ref[1]
    sblk = jnp.concatenate([sa, sb], axis=1)
    t = jnp.dot(sblk, w1a_ref[...], preferred_element_type=jnp.float32)
    t = jnp.maximum(t + b1a_ref[...], 0.0)
    u = jnp.dot(t, w1b_ref[...], preferred_element_type=jnp.float32)
    u = jnp.maximum(u + b1b_ref[...], 0.0)
    nrm = jnp.sqrt(jnp.sum(u * u, axis=1, keepdims=True))
    h = u / jnp.maximum(nrm, 1e-12)
    v = jnp.dot(h, wl1_ref[...], preferred_element_type=jnp.float32) + bl1_ref[...]
    v = jnp.where(v > 0.0, v, jnp.exp(v) - 1.0)
    out_ref[...] = jnp.dot(v, wl2_ref[...], preferred_element_type=jnp.float32) + bl2_ref[...]


_R = 1000


def _row_spec(shape_cols):
    return pl.BlockSpec((_R, shape_cols), lambda i: (i, 0))


def _part_spec():
    return pl.BlockSpec((2, _R, D), lambda i: (0, i, 0))


def _full_spec(r, c):
    return pl.BlockSpec((r, c), lambda i: (0, 0))


def _tc1(x, p0, wa, ba, wb, bb):
    return pl.pallas_call(
        _tc1_body,
        grid=(N // _R,),
        in_specs=[
            _row_spec(D), _part_spec(),
            _full_spec(128, 256), _full_spec(1, 256),
            _full_spec(256, 256), _full_spec(1, 256),
        ],
        out_specs=[_row_spec(D), _row_spec(D)],
        out_shape=[jax.ShapeDtypeStruct((N, D), jnp.float32)] * 2,
    )(x, p0, wa, ba, wb, bb)


def _tc2(ha, hb, pa, pb, w1a, b1a, w1b, b1b, wl1, bl1, wl2, bl2):
    return pl.pallas_call(
        _tc2_body,
        grid=(N // _R,),
        in_specs=[
            _row_spec(D), _row_spec(D), _part_spec(), _part_spec(),
            _full_spec(256, 256), _full_spec(1, 256),
            _full_spec(256, 256), _full_spec(1, 256),
            _full_spec(256, 256), _full_spec(1, 256),
            _full_spec(256, 256), _full_spec(1, 256),
        ],
        out_specs=pl.BlockSpec((_R, 256), lambda i: (i, 0)),
        out_shape=jax.ShapeDtypeStruct((N, 256), jnp.float32),
    )(ha, hb, pa, pb, w1a, b1a, w1b, b1b, wl1, bl1, wl2, bl2)


def kernel(x, edges, W0a, b0a, W0b, b0b, W1a, b1a, W1b, b1b, Wl1, bl1, Wl2, bl2):
    src = edges[0]
    dst = edges[1]
    pad_dst = N + jnp.arange(EPAD - E, dtype=jnp.int32) % (NPAD - N)
    srcp = jnp.concatenate([src, jnp.zeros((EPAD - E,), jnp.int32)])
    dstp = jnp.concatenate([dst, pad_dst])
    src3 = srcp.reshape(EPAD // (SPAN * K), SPAN, K)
    dst3 = dstp.reshape(EPAD // (SPAN * K), SPAN, K)
    zeros = jnp.zeros((NPAD, D), jnp.float32)

    agg = _make_sc_aggregate(2, 2)
    p0 = agg(x, src3, dst3, zeros)
    h1a, h1b = _tc1(x, p0, W0a, b0a.reshape(1, -1), W0b, b0b.reshape(1, -1))
    pa = agg(h1a, src3, dst3, zeros)
    pb = agg(h1b, src3, dst3, zeros)
    return _tc2(h1a, h1b, pa, pb,
                W1a, b1a.reshape(1, -1), W1b, b1b.reshape(1, -1),
                Wl1, bl1.reshape(1, -1), Wl2, bl2.reshape(1, -1))


# speedup vs baseline: 3.1503x; 3.1503x over previous
import functools

import jax
import jax.numpy as jnp
from jax import lax
from jax.experimental import pallas as pl
from jax.experimental.pallas import tpu as pltpu
from jax.experimental.pallas import tpu_sc as plsc

N = 10000
E = 320000
D = 128
K = 128
EPAD = 327680
NPAD = 10368
RPT = NPAD // 16
NBUF = 2
LOOK = 1
SPAN = 40


@functools.lru_cache(maxsize=None)
def _make_sc_aggregate(heavy_spans, light_spans):
    slots = (heavy_spans + light_spans) * 16
    assert slots * SPAN * K == EPAD
    mesh = plsc.VectorSubcoreMesh(core_axis_name="c", subcore_axis_name="s")

    @functools.partial(
        pl.kernel,
        out_type=jax.ShapeDtypeStruct((2, NPAD, D), jnp.float32),
        mesh=mesh,
        scratch_types=[
            pltpu.VMEM((SPAN, K), jnp.int32),
            pltpu.VMEM((SPAN, K), jnp.int32),
            [pltpu.VMEM((K, D), jnp.float32)] * NBUF,
            pltpu.VMEM_SHARED((NPAD, D), jnp.float32),
            [pltpu.SemaphoreType.DMA] * NBUF,
            [pltpu.SemaphoreType.DMA] * NBUF,
        ],
    )
    def sc_aggregate(table, src3, dst3, zeros, out, src_v, dst_v, bufs, acc,
                     gsems, ssems):
        c = lax.axis_index("c")
        s = lax.axis_index("s")
        tab = table
        rows = pl.ds(s * RPT, RPT)
        with jax.named_scope("zinit"):
            pltpu.sync_copy(zeros.at[rows], acc.at[rows])
            plsc.subcore_barrier()

        def wait_gather(j, b):
            pltpu.make_async_copy(tab.at[src_v.at[j]], bufs[b],
                                  gsems[b]).wait()

        def fire_scatter(j, b):
            pltpu.async_copy(bufs[b], acc.at[dst_v.at[j]], ssems[b], add=True)

        def wait_scatter(j, b):
            pltpu.make_async_copy(bufs[b], acc.at[dst_v.at[j]],
                                  ssems[b]).wait()

        def fire_gather(j, b):
            pltpu.async_copy(tab.at[src_v.at[j]], bufs[b], gsems[b])

        def steady(j, b):
            wait_gather(j, b)
            fire_scatter(j, b)
            bn = (b + LOOK) % NBUF
            wait_scatter(j + LOOK - NBUF, bn)
            fire_gather(j + LOOK, bn)

        def run_span(slot):
            pltpu.sync_copy(src3.at[slot], src_v)
            pltpu.sync_copy(dst3.at[slot], dst_v)
            for b in range(LOOK):
                pltpu.async_copy(tab.at[src_v.at[b]], bufs[b], gsems[b])

            for j in range(NBUF - LOOK):
                wait_gather(j, j)
                fire_scatter(j, j)
                fire_gather(j + LOOK, j + LOOK)

            start = NBUF - LOOK
            rem = (SPAN - NBUF) % NBUF
            for j in range(start, start + rem):
                steady(j, j % NBUF)

            def group(g, carry):
                for i in range(NBUF):
                    steady(start + rem + g * NBUF + i,
                           (start + rem + i) % NBUF)
                return carry

            lax.fori_loop(0, (SPAN - NBUF) // NBUF, group, 0)

            for j in range(SPAN - LOOK, SPAN):
                wait_gather(j, j % NBUF)
                fire_scatter(j, j % NBUF)
            for m in range(SPAN - NBUF, SPAN):
                wait_scatter(m, m % NBUF)

        with jax.named_scope("edges"):
            @pl.when(c == 0)
            def _():
                for sp in range(heavy_spans):
                    run_span(sp * 16 + s)

            @pl.when(c == 1)
            def _():
                for sp in range(light_spans):
                    run_span((heavy_spans + sp) * 16 + s)

        with jax.named_scope("outbar"):
            plsc.subcore_barrier()
        with jax.named_scope("outcp"):
            pltpu.sync_copy(acc.at[rows], out.at[c, rows])

    return sc_aggregate


def _tc1_body(x_ref, p_ref, wa_ref, ba_ref, wb_ref, bb_ref, ha_ref, hb_ref):
    sblk = x_ref[...] + p_ref[0] + p_ref[1]
    t = jnp.dot(sblk, wa_ref[...], preferred_element_type=jnp.float32)
    t = jnp.maximum(t + ba_ref[...], 0.0)
    u = jnp.dot(t, wb_ref[...], preferred_element_type=jnp.float32)
    u = jnp.maximum(u + bb_ref[...], 0.0)
    nrm = jnp.sqrt(jnp.sum(u * u, axis=1, keepdims=True))
    h = u / jnp.maximum(nrm, 1e-12)
    ha_ref[...] = h[:, :D]
    hb_ref[...] = h[:, D:]


def _tc2_body(ha_ref, hb_ref, pa_ref, pb_ref, w1a_ref, b1a_ref, w1b_ref,
              b1b_ref, wl1_ref, bl1_ref, wl2_ref, bl2_ref, out_ref):
    sa = ha_ref[...] + pa_ref[0] + pa_ref[1]
    sb = hb_ref[...] + pb_ref[0] + pb_ref[1]
    sblk = jnp.concatenate([sa, sb], axis=1)
    t = jnp.dot(sblk, w1a_ref[...], preferred_element_type=jnp.float32)
    t = jnp.maximum(t + b1a_ref[...], 0.0)
    u = jnp.dot(t, w1b_ref[...], preferred_element_type=jnp.float32)
    u = jnp.maximum(u + b1b_ref[...], 0.0)
    nrm = jnp.sqrt(jnp.sum(u * u, axis=1, keepdims=True))
    h = u / jnp.maximum(nrm, 1e-12)
    v = jnp.dot(h, wl1_ref[...], preferred_element_type=jnp.float32) + bl1_ref[...]
    v = jnp.where(v > 0.0, v, jnp.exp(v) - 1.0)
    out_ref[...] = jnp.dot(v, wl2_ref[...], preferred_element_type=jnp.float32) + bl2_ref[...]


_R = 1000


def _row_spec(shape_cols):
    return pl.BlockSpec((_R, shape_cols), lambda i: (i, 0))


def _part_spec():
    return pl.BlockSpec((2, _R, D), lambda i: (0, i, 0))


def _full_spec(r, c):
    return pl.BlockSpec((r, c), lambda i: (0, 0))


def _tc1(x, p0, wa, ba, wb, bb):
    return pl.pallas_call(
        _tc1_body,
        grid=(N // _R,),
        in_specs=[
            _row_spec(D), _part_spec(),
            _full_spec(128, 256), _full_spec(1, 256),
            _full_spec(256, 256), _full_spec(1, 256),
        ],
        out_specs=[_row_spec(D), _row_spec(D)],
        out_shape=[jax.ShapeDtypeStruct((N, D), jnp.float32)] * 2,
    )(x, p0, wa, ba, wb, bb)


def _tc2(ha, hb, pa, pb, w1a, b1a, w1b, b1b, wl1, bl1, wl2, bl2):
    return pl.pallas_call(
        _tc2_body,
        grid=(N // _R,),
        in_specs=[
            _row_spec(D), _row_spec(D), _part_spec(), _part_spec(),
            _full_spec(256, 256), _full_spec(1, 256),
            _full_spec(256, 256), _full_spec(1, 256),
            _full_spec(256, 256), _full_spec(1, 256),
            _full_spec(256, 256), _full_spec(1, 256),
        ],
        out_specs=pl.BlockSpec((_R, 256), lambda i: (i, 0)),
        out_shape=jax.ShapeDtypeStruct((N, 256), jnp.float32),
    )(ha, hb, pa, pb, w1a, b1a, w1b, b1b, wl1, bl1, wl2, bl2)


def kernel(x, edges, W0a, b0a, W0b, b0b, W1a, b1a, W1b, b1b, Wl1, bl1, Wl2, bl2):
    src = edges[0]
    dst = edges[1]
    pad_i = jnp.arange(EPAD - E, dtype=jnp.int32)
    srcp = jnp.concatenate([src, pad_i % N])
    dstp = jnp.concatenate([dst, N + pad_i % (NPAD - N)])
    src3 = srcp.reshape(EPAD // (SPAN * K), SPAN, K)
    dst3 = dstp.reshape(EPAD // (SPAN * K), SPAN, K)
    zeros = jnp.zeros((NPAD, D), jnp.float32)

    agg = _make_sc_aggregate(2, 2)
    p0 = agg(x, src3, dst3, zeros)
    h1a, h1b = _tc1(x, p0, W0a, b0a.reshape(1, -1), W0b, b0b.reshape(1, -1))
    pa = agg(h1a, src3, dst3, zeros)
    pb = agg(h1b, src3, dst3, zeros)
    return _tc2(h1a, h1b, pa, pb,
                W1a, b1a.reshape(1, -1), W1b, b1b.reshape(1, -1),
                Wl1, bl1.reshape(1, -1), Wl2, bl2.reshape(1, -1))


# speedup vs baseline: 3.2055x; 1.0175x over previous
import functools

import jax
import jax.numpy as jnp
from jax import lax
from jax.experimental import pallas as pl
from jax.experimental.pallas import tpu as pltpu
from jax.experimental.pallas import tpu_sc as plsc

N = 10000
E = 320000
D = 128
K = 64
EPAD = 327680
NPAD = 10368
RPT = NPAD // 16
NBUF = 4
LOOK = 2
SPAN = 40


@functools.lru_cache(maxsize=None)
def _make_sc_aggregate(heavy_spans, light_spans):
    slots = (heavy_spans + light_spans) * 16
    assert slots * SPAN * K == EPAD
    mesh = plsc.VectorSubcoreMesh(core_axis_name="c", subcore_axis_name="s")

    @functools.partial(
        pl.kernel,
        out_type=jax.ShapeDtypeStruct((2, NPAD, D), jnp.float32),
        mesh=mesh,
        scratch_types=[
            pltpu.VMEM((SPAN, K), jnp.int32),
            pltpu.VMEM((SPAN, K), jnp.int32),
            [pltpu.VMEM((K, D), jnp.float32)] * NBUF,
            pltpu.VMEM_SHARED((NPAD, D), jnp.float32),
            [pltpu.SemaphoreType.DMA] * NBUF,
            [pltpu.SemaphoreType.DMA] * NBUF,
        ],
    )
    def sc_aggregate(table, src3, dst3, zeros, out, src_v, dst_v, bufs, acc,
                     gsems, ssems):
        c = lax.axis_index("c")
        s = lax.axis_index("s")
        tab = table
        rows = pl.ds(s * RPT, RPT)
        with jax.named_scope("zinit"):
            pltpu.sync_copy(zeros.at[rows], acc.at[rows])
            plsc.subcore_barrier()

        def wait_gather(j, b):
            pltpu.make_async_copy(tab.at[src_v.at[j]], bufs[b],
                                  gsems[b]).wait()

        def fire_scatter(j, b):
            pltpu.async_copy(bufs[b], acc.at[dst_v.at[j]], ssems[b], add=True)

        def wait_scatter(j, b):
            pltpu.make_async_copy(bufs[b], acc.at[dst_v.at[j]],
                                  ssems[b]).wait()

        def fire_gather(j, b):
            pltpu.async_copy(tab.at[src_v.at[j]], bufs[b], gsems[b])

        def steady(j, b):
            wait_gather(j, b)
            fire_scatter(j, b)
            bn = (b + LOOK) % NBUF
            wait_scatter(j + LOOK - NBUF, bn)
            fire_gather(j + LOOK, bn)

        def run_span(slot):
            pltpu.sync_copy(src3.at[slot], src_v)
            pltpu.sync_copy(dst3.at[slot], dst_v)
            for b in range(LOOK):
                pltpu.async_copy(tab.at[src_v.at[b]], bufs[b], gsems[b])

            for j in range(NBUF - LOOK):
                wait_gather(j, j)
                fire_scatter(j, j)
                fire_gather(j + LOOK, j + LOOK)

            start = NBUF - LOOK
            rem = (SPAN - NBUF) % NBUF
            for j in range(start, start + rem):
                steady(j, j % NBUF)

            def group(g, carry):
                for i in range(NBUF):
                    steady(start + rem + g * NBUF + i,
                           (start + rem + i) % NBUF)
                return carry

            lax.fori_loop(0, (SPAN - NBUF) // NBUF, group, 0)

            for j in range(SPAN - LOOK, SPAN):
                wait_gather(j, j % NBUF)
                fire_scatter(j, j % NBUF)
            for m in range(SPAN - NBUF, SPAN):
                wait_scatter(m, m % NBUF)

        with jax.named_scope("edges"):
            @pl.when(c == 0)
            def _():
                for sp in range(heavy_spans):
                    run_span(sp * 16 + s)

            @pl.when(c == 1)
            def _():
                for sp in range(light_spans):
                    run_span((heavy_spans + sp) * 16 + s)

        with jax.named_scope("outbar"):
            plsc.subcore_barrier()
        with jax.named_scope("outcp"):
            pltpu.sync_copy(acc.at[rows], out.at[c, rows])

    return sc_aggregate


def _tc1_body(x_ref, p_ref, wa_ref, ba_ref, wb_ref, bb_ref, ha_ref, hb_ref):
    sblk = x_ref[...] + p_ref[0] + p_ref[1]
    t = jnp.dot(sblk, wa_ref[...], preferred_element_type=jnp.float32)
    t = jnp.maximum(t + ba_ref[...], 0.0)
    u = jnp.dot(t, wb_ref[...], preferred_element_type=jnp.float32)
    u = jnp.maximum(u + bb_ref[...], 0.0)
    nrm = jnp.sqrt(jnp.sum(u * u, axis=1, keepdims=True))
    h = u / jnp.maximum(nrm, 1e-12)
    ha_ref[...] = h[:, :D]
    hb_ref[...] = h[:, D:]


def _tc2_body(ha_ref, hb_ref, pa_ref, pb_ref, w1a_ref, b1a_ref, w1b_ref,
              b1b_ref, wl1_ref, bl1_ref, wl2_ref, bl2_ref, out_ref):
    sa = ha_ref[...] + pa_ref[0] + pa_ref[1]
    sb = hb_ref[...] + pb_ref[0] + pb_ref[1]
    sblk = jnp.concatenate([sa, sb], axis=1)
    t = jnp.dot(sblk, w1a_ref[...], preferred_element_type=jnp.float32)
    t = jnp.maximum(t + b1a_ref[...], 0.0)
    u = jnp.dot(t, w1b_ref[...], preferred_element_type=jnp.float32)
    u = jnp.maximum(u + b1b_ref[...], 0.0)
    nrm = jnp.sqrt(jnp.sum(u * u, axis=1, keepdims=True))
    h = u / jnp.maximum(nrm, 1e-12)
    v = jnp.dot(h, wl1_ref[...], preferred_element_type=jnp.float32) + bl1_ref[...]
    v = jnp.where(v > 0.0, v, jnp.exp(v) - 1.0)
    out_ref[...] = jnp.dot(v, wl2_ref[...], preferred_element_type=jnp.float32) + bl2_ref[...]


_R = 1000


def _row_spec(shape_cols):
    return pl.BlockSpec((_R, shape_cols), lambda i: (i, 0))


def _part_spec():
    return pl.BlockSpec((2, _R, D), lambda i: (0, i, 0))


def _full_spec(r, c):
    return pl.BlockSpec((r, c), lambda i: (0, 0))


def _tc1(x, p0, wa, ba, wb, bb):
    return pl.pallas_call(
        _tc1_body,
        grid=(N // _R,),
        in_specs=[
            _row_spec(D), _part_spec(),
            _full_spec(128, 256), _full_spec(1, 256),
            _full_spec(256, 256), _full_spec(1, 256),
        ],
        out_specs=[_row_spec(D), _row_spec(D)],
        out_shape=[jax.ShapeDtypeStruct((N, D), jnp.float32)] * 2,
    )(x, p0, wa, ba, wb, bb)


def _tc2(ha, hb, pa, pb, w1a, b1a, w1b, b1b, wl1, bl1, wl2, bl2):
    return pl.pallas_call(
        _tc2_body,
        grid=(N // _R,),
        in_specs=[
            _row_spec(D), _row_spec(D), _part_spec(), _part_spec(),
            _full_spec(256, 256), _full_spec(1, 256),
            _full_spec(256, 256), _full_spec(1, 256),
            _full_spec(256, 256), _full_spec(1, 256),
            _full_spec(256, 256), _full_spec(1, 256),
        ],
        out_specs=pl.BlockSpec((_R, 256), lambda i: (i, 0)),
        out_shape=jax.ShapeDtypeStruct((N, 256), jnp.float32),
    )(ha, hb, pa, pb, w1a, b1a, w1b, b1b, wl1, bl1, wl2, bl2)


def kernel(x, edges, W0a, b0a, W0b, b0b, W1a, b1a, W1b, b1b, Wl1, bl1, Wl2, bl2):
    src = edges[0]
    dst = edges[1]
    pad_i = jnp.arange(EPAD - E, dtype=jnp.int32)
    srcp = jnp.concatenate([src, pad_i % N])
    dstp = jnp.concatenate([dst, N + pad_i % (NPAD - N)])
    src3 = srcp.reshape(EPAD // (SPAN * K), SPAN, K)
    dst3 = dstp.reshape(EPAD // (SPAN * K), SPAN, K)
    zeros = jnp.zeros((NPAD, D), jnp.float32)

    agg = _make_sc_aggregate(4, 4)
    p0 = agg(x, src3, dst3, zeros)
    h1a, h1b = _tc1(x, p0, W0a, b0a.reshape(1, -1), W0b, b0b.reshape(1, -1))
    pa = agg(h1a, src3, dst3, zeros)
    pb = agg(h1b, src3, dst3, zeros)
    return _tc2(h1a, h1b, pa, pb,
                W1a, b1a.reshape(1, -1), W1b, b1b.reshape(1, -1),
                Wl1, bl1.reshape(1, -1), Wl2, bl2.reshape(1, -1))


# speedup vs baseline: 3.3715x; 1.0518x over previous
import functools

import jax
import jax.numpy as jnp
from jax import lax
from jax.experimental import pallas as pl
from jax.experimental.pallas import tpu as pltpu
from jax.experimental.pallas import tpu_sc as plsc

N = 10000
E = 320000
D = 128
K = 64
EPAD = 327680
NPAD = 10368
RPT = NPAD // 16
NBUF = 4
LOOK = 2
SPAN = 40


@functools.lru_cache(maxsize=None)
def _make_sc_aggregate(heavy_spans, light_spans, stacked=False):
    slots = (heavy_spans if stacked else heavy_spans + light_spans) * 16
    assert slots * SPAN * K == EPAD
    mesh = plsc.VectorSubcoreMesh(core_axis_name="c", subcore_axis_name="s")

    @functools.partial(
        pl.kernel,
        out_type=jax.ShapeDtypeStruct((2, NPAD, D), jnp.float32),
        mesh=mesh,
        scratch_types=[
            pltpu.VMEM((SPAN, K), jnp.int32),
            pltpu.VMEM((SPAN, K), jnp.int32),
            [pltpu.VMEM((K, D), jnp.float32)] * NBUF,
            pltpu.VMEM_SHARED((NPAD, D), jnp.float32),
            [pltpu.SemaphoreType.DMA] * NBUF,
            [pltpu.SemaphoreType.DMA] * NBUF,
        ],
    )
    def sc_aggregate(table, src3, dst3, zeros, out, src_v, dst_v, bufs, acc,
                     gsems, ssems):
        c = lax.axis_index("c")
        s = lax.axis_index("s")
        tab = table.at[c] if stacked else table
        rows = pl.ds(s * RPT, RPT)
        with jax.named_scope("zinit"):
            pltpu.sync_copy(zeros.at[rows], acc.at[rows])
            plsc.subcore_barrier()

        def wait_gather(j, b):
            pltpu.make_async_copy(tab.at[src_v.at[j]], bufs[b],
                                  gsems[b]).wait()

        def fire_scatter(j, b):
            pltpu.async_copy(bufs[b], acc.at[dst_v.at[j]], ssems[b], add=True)

        def wait_scatter(j, b):
            pltpu.make_async_copy(bufs[b], acc.at[dst_v.at[j]],
                                  ssems[b]).wait()

        def fire_gather(j, b):
            pltpu.async_copy(tab.at[src_v.at[j]], bufs[b], gsems[b])

        def steady(j, b):
            wait_gather(j, b)
            fire_scatter(j, b)
            bn = (b + LOOK) % NBUF
            wait_scatter(j + LOOK - NBUF, bn)
            fire_gather(j + LOOK, bn)

        def run_span(slot):
            pltpu.sync_copy(src3.at[slot], src_v)
            pltpu.sync_copy(dst3.at[slot], dst_v)
            for b in range(LOOK):
                pltpu.async_copy(tab.at[src_v.at[b]], bufs[b], gsems[b])

            for j in range(NBUF - LOOK):
                wait_gather(j, j)
                fire_scatter(j, j)
                fire_gather(j + LOOK, j + LOOK)

            start = NBUF - LOOK
            rem = (SPAN - NBUF) % NBUF
            for j in range(start, start + rem):
                steady(j, j % NBUF)

            def group(g, carry):
                for i in range(NBUF):
                    steady(start + rem + g * NBUF + i,
                           (start + rem + i) % NBUF)
                return carry

            lax.fori_loop(0, (SPAN - NBUF) // NBUF, group, 0)

            for j in range(SPAN - LOOK, SPAN):
                wait_gather(j, j % NBUF)
                fire_scatter(j, j % NBUF)
            for m in range(SPAN - NBUF, SPAN):
                wait_scatter(m, m % NBUF)

        with jax.named_scope("edges"):
            if stacked:
                for sp in range(heavy_spans):
                    run_span(sp * 16 + s)
            else:
                @pl.when(c == 0)
                def _():
                    for sp in range(heavy_spans):
                        run_span(sp * 16 + s)

                @pl.when(c == 1)
                def _():
                    for sp in range(light_spans):
                        run_span((heavy_spans + sp) * 16 + s)

        with jax.named_scope("outbar"):
            plsc.subcore_barrier()
        with jax.named_scope("outcp"):
            pltpu.sync_copy(acc.at[rows], out.at[c, rows])

    return sc_aggregate


def _tc1_body(x_ref, p_ref, wa_ref, ba_ref, wb_ref, bb_ref, h_ref):
    sblk = x_ref[...] + p_ref[0] + p_ref[1]
    t = jnp.dot(sblk, wa_ref[...], preferred_element_type=jnp.float32)
    t = jnp.maximum(t + ba_ref[...], 0.0)
    u = jnp.dot(t, wb_ref[...], preferred_element_type=jnp.float32)
    u = jnp.maximum(u + bb_ref[...], 0.0)
    nrm = jnp.sqrt(jnp.sum(u * u, axis=1, keepdims=True))
    h = u / jnp.maximum(nrm, 1e-12)
    h_ref[...] = jnp.stack([h[:, :D], h[:, D:]])


def _tc2_body(h_ref, p_ref, w1a_ref, b1a_ref, w1b_ref,
              b1b_ref, wl1_ref, bl1_ref, wl2_ref, bl2_ref, out_ref):
    sa = h_ref[0] + p_ref[0]
    sb = h_ref[1] + p_ref[1]
    sblk = jnp.concatenate([sa, sb], axis=1)
    t = jnp.dot(sblk, w1a_ref[...], preferred_element_type=jnp.float32)
    t = jnp.maximum(t + b1a_ref[...], 0.0)
    u = jnp.dot(t, w1b_ref[...], preferred_element_type=jnp.float32)
    u = jnp.maximum(u + b1b_ref[...], 0.0)
    nrm = jnp.sqrt(jnp.sum(u * u, axis=1, keepdims=True))
    h = u / jnp.maximum(nrm, 1e-12)
    v = jnp.dot(h, wl1_ref[...], preferred_element_type=jnp.float32) + bl1_ref[...]
    v = jnp.where(v > 0.0, v, jnp.exp(v) - 1.0)
    out_ref[...] = jnp.dot(v, wl2_ref[...], preferred_element_type=jnp.float32) + bl2_ref[...]


_R = 1000


def _row_spec(shape_cols):
    return pl.BlockSpec((_R, shape_cols), lambda i: (i, 0))


def _part_spec():
    return pl.BlockSpec((2, _R, D), lambda i: (0, i, 0))


def _full_spec(r, c):
    return pl.BlockSpec((r, c), lambda i: (0, 0))


def _tc1(x, p0, wa, ba, wb, bb):
    return pl.pallas_call(
        _tc1_body,
        grid=(N // _R,),
        in_specs=[
            _row_spec(D), _part_spec(),
            _full_spec(128, 256), _full_spec(1, 256),
            _full_spec(256, 256), _full_spec(1, 256),
        ],
        out_specs=_part_spec(),
        out_shape=jax.ShapeDtypeStruct((2, N, D), jnp.float32),
    )(x, p0, wa, ba, wb, bb)


def _tc2(h1s, p1, w1a, b1a, w1b, b1b, wl1, bl1, wl2, bl2):
    return pl.pallas_call(
        _tc2_body,
        grid=(N // _R,),
        in_specs=[
            _part_spec(), _part_spec(),
            _full_spec(256, 256), _full_spec(1, 256),
            _full_spec(256, 256), _full_spec(1, 256),
            _full_spec(256, 256), _full_spec(1, 256),
            _full_spec(256, 256), _full_spec(1, 256),
        ],
        out_specs=pl.BlockSpec((_R, 256), lambda i: (i, 0)),
        out_shape=jax.ShapeDtypeStruct((N, 256), jnp.float32),
    )(h1s, p1, w1a, b1a, w1b, b1b, wl1, bl1, wl2, bl2)


def kernel(x, edges, W0a, b0a, W0b, b0b, W1a, b1a, W1b, b1b, Wl1, bl1, Wl2, bl2):
    src = edges[0]
    dst = edges[1]
    pad_i = jnp.arange(EPAD - E, dtype=jnp.int32)
    srcp = jnp.concatenate([src, pad_i % N])
    dstp = jnp.concatenate([dst, N + pad_i % (NPAD - N)])
    src3 = srcp.reshape(EPAD // (SPAN * K), SPAN, K)
    dst3 = dstp.reshape(EPAD // (SPAN * K), SPAN, K)
    zeros = jnp.zeros((NPAD, D), jnp.float32)

    p0 = _make_sc_aggregate(4, 4)(x, src3, dst3, zeros)
    h1s = _tc1(x, p0, W0a, b0a.reshape(1, -1), W0b, b0b.reshape(1, -1))
    p1 = _make_sc_aggregate(8, 0, True)(h1s, src3, dst3, zeros)
    return _tc2(h1s, p1,
                W1a, b1a.reshape(1, -1), W1b, b1b.reshape(1, -1),
                Wl1, bl1.reshape(1, -1), Wl2, bl2.reshape(1, -1))


# speedup vs baseline: 3.3729x; 1.0004x over previous
import functools

import jax
import jax.numpy as jnp
from jax import lax
from jax.experimental import pallas as pl
from jax.experimental.pallas import tpu as pltpu
from jax.experimental.pallas import tpu_sc as plsc

N = 10000
E = 320000
D = 128
K = 64
EPAD = 327680
NPAD = 10368
RPT = NPAD // 16
NBUF = 4
LOOK = 2
SPAN = 40


@functools.lru_cache(maxsize=None)
def _make_sc_aggregate(heavy_spans, light_spans, stacked=False):
    slots = (heavy_spans if stacked else heavy_spans + light_spans) * 16
    assert slots * SPAN * K == EPAD
    mesh = plsc.VectorSubcoreMesh(core_axis_name="c", subcore_axis_name="s")

    @functools.partial(
        pl.kernel,
        out_type=jax.ShapeDtypeStruct((2, NPAD, D), jnp.float32),
        mesh=mesh,
        scratch_types=[
            pltpu.VMEM((SPAN, K), jnp.int32),
            pltpu.VMEM((SPAN, K), jnp.int32),
            [pltpu.VMEM((K, D), jnp.float32)] * NBUF,
            pltpu.VMEM_SHARED((NPAD, D), jnp.float32),
            [pltpu.SemaphoreType.DMA] * NBUF,
            [pltpu.SemaphoreType.DMA] * NBUF,
        ],
    )
    def sc_aggregate(table, src3, dst3, zeros, out, src_v, dst_v, bufs, acc,
                     gsems, ssems):
        c = lax.axis_index("c")
        s = lax.axis_index("s")
        tab = table.at[c] if stacked else table
        rows = pl.ds(s * RPT, RPT)
        pltpu.sync_copy(zeros.at[rows], acc.at[rows])
        plsc.subcore_barrier()

        def wait_gather(j, b):
            pltpu.make_async_copy(tab.at[src_v.at[j]], bufs[b],
                                  gsems[b]).wait()

        def fire_scatter(j, b):
            pltpu.async_copy(bufs[b], acc.at[dst_v.at[j]], ssems[b], add=True)

        def wait_scatter(j, b):
            pltpu.make_async_copy(bufs[b], acc.at[dst_v.at[j]],
                                  ssems[b]).wait()

        def fire_gather(j, b):
            pltpu.async_copy(tab.at[src_v.at[j]], bufs[b], gsems[b])

        def steady(j, b):
            wait_gather(j, b)
            fire_scatter(j, b)
            bn = (b + LOOK) % NBUF
            wait_scatter(j + LOOK - NBUF, bn)
            fire_gather(j + LOOK, bn)

        def run_span(slot):
            pltpu.sync_copy(src3.at[slot], src_v)
            pltpu.sync_copy(dst3.at[slot], dst_v)
            for b in range(LOOK):
                pltpu.async_copy(tab.at[src_v.at[b]], bufs[b], gsems[b])

            for j in range(NBUF - LOOK):
                wait_gather(j, j)
                fire_scatter(j, j)
                fire_gather(j + LOOK, j + LOOK)

            start = NBUF - LOOK
            rem = (SPAN - NBUF) % NBUF
            for j in range(start, start + rem):
                steady(j, j % NBUF)

            def group(g, carry):
                for i in range(NBUF):
                    steady(start + rem + g * NBUF + i,
                           (start + rem + i) % NBUF)
                return carry

            lax.fori_loop(0, (SPAN - NBUF) // NBUF, group, 0)

            for j in range(SPAN - LOOK, SPAN):
                wait_gather(j, j % NBUF)
                fire_scatter(j, j % NBUF)
            for m in range(SPAN - NBUF, SPAN):
                wait_scatter(m, m % NBUF)

        if stacked:
            for sp in range(heavy_spans):
                run_span(sp * 16 + s)
        else:
            @pl.when(c == 0)
            def _():
                for sp in range(heavy_spans):
                    run_span(sp * 16 + s)

            @pl.when(c == 1)
            def _():
                for sp in range(light_spans):
                    run_span((heavy_spans + sp) * 16 + s)

        plsc.subcore_barrier()
        pltpu.sync_copy(acc.at[rows], out.at[c, rows])

    return sc_aggregate


def _tc1_body(x_ref, p_ref, wa_ref, ba_ref, wb_ref, bb_ref, h_ref):
    sblk = x_ref[...] + p_ref[0] + p_ref[1]
    t = jnp.dot(sblk, wa_ref[...], preferred_element_type=jnp.float32)
    t = jnp.maximum(t + ba_ref[...], 0.0)
    u = jnp.dot(t, wb_ref[...], preferred_element_type=jnp.float32)
    u = jnp.maximum(u + bb_ref[...], 0.0)
    nrm = jnp.sqrt(jnp.sum(u * u, axis=1, keepdims=True))
    h = u / jnp.maximum(nrm, 1e-12)
    h_ref[...] = jnp.stack([h[:, :D], h[:, D:]])


def _tc2_body(h_ref, p_ref, w1a_ref, b1a_ref, w1b_ref,
              b1b_ref, wl1_ref, bl1_ref, wl2_ref, bl2_ref, out_ref):
    sa = h_ref[0] + p_ref[0]
    sb = h_ref[1] + p_ref[1]
    sblk = jnp.concatenate([sa, sb], axis=1)
    t = jnp.dot(sblk, w1a_ref[...], preferred_element_type=jnp.float32)
    t = jnp.maximum(t + b1a_ref[...], 0.0)
    u = jnp.dot(t, w1b_ref[...], preferred_element_type=jnp.float32)
    u = jnp.maximum(u + b1b_ref[...], 0.0)
    nrm = jnp.sqrt(jnp.sum(u * u, axis=1, keepdims=True))
    h = u / jnp.maximum(nrm, 1e-12)
    v = jnp.dot(h, wl1_ref[...], preferred_element_type=jnp.float32) + bl1_ref[...]
    v = jnp.where(v > 0.0, v, jnp.exp(v) - 1.0)
    out_ref[...] = jnp.dot(v, wl2_ref[...], preferred_element_type=jnp.float32) + bl2_ref[...]


_R = 1000


def _row_spec(shape_cols):
    return pl.BlockSpec((_R, shape_cols), lambda i: (i, 0))


def _part_spec():
    return pl.BlockSpec((2, _R, D), lambda i: (0, i, 0))


def _full_spec(r, c):
    return pl.BlockSpec((r, c), lambda i: (0, 0))


def _tc1(x, p0, wa, ba, wb, bb):
    return pl.pallas_call(
        _tc1_body,
        grid=(N // _R,),
        in_specs=[
            _row_spec(D), _part_spec(),
            _full_spec(128, 256), _full_spec(1, 256),
            _full_spec(256, 256), _full_spec(1, 256),
        ],
        out_specs=_part_spec(),
        out_shape=jax.ShapeDtypeStruct((2, N, D), jnp.float32),
    )(x, p0, wa, ba, wb, bb)


def _tc2(h1s, p1, w1a, b1a, w1b, b1b, wl1, bl1, wl2, bl2):
    return pl.pallas_call(
        _tc2_body,
        grid=(N // _R,),
        in_specs=[
            _part_spec(), _part_spec(),
            _full_spec(256, 256), _full_spec(1, 256),
            _full_spec(256, 256), _full_spec(1, 256),
            _full_spec(256, 256), _full_spec(1, 256),
            _full_spec(256, 256), _full_spec(1, 256),
        ],
        out_specs=pl.BlockSpec((_R, 256), lambda i: (i, 0)),
        out_shape=jax.ShapeDtypeStruct((N, 256), jnp.float32),
    )(h1s, p1, w1a, b1a, w1b, b1b, wl1, bl1, wl2, bl2)


def kernel(x, edges, W0a, b0a, W0b, b0b, W1a, b1a, W1b, b1b, Wl1, bl1, Wl2, bl2):
    src = edges[0]
    dst = edges[1]
    pad_i = jnp.arange(EPAD - E, dtype=jnp.int32)
    srcp = jnp.concatenate([src, pad_i % N])
    dstp = jnp.concatenate([dst, N + pad_i % (NPAD - N)])
    src3 = srcp.reshape(EPAD // (SPAN * K), SPAN, K)
    dst3 = dstp.reshape(EPAD // (SPAN * K), SPAN, K)
    zeros = jnp.zeros((NPAD, D), jnp.float32)

    p0 = _make_sc_aggregate(4, 4)(x, src3, dst3, zeros)
    h1s = _tc1(x, p0, W0a, b0a.reshape(1, -1), W0b, b0b.reshape(1, -1))
    p1 = _make_sc_aggregate(8, 0, True)(h1s, src3, dst3, zeros)
    return _tc2(h1s, p1,
                W1a, b1a.reshape(1, -1), W1b, b1b.reshape(1, -1),
                Wl1, bl1.reshape(1, -1), Wl2, bl2.reshape(1, -1))
